# baseline (speedup 1.0000x reference)
import functools

import jax
import jax.numpy as jnp
from jax import lax
from jax.experimental import pallas as pl
from jax.experimental.pallas import tpu as pltpu
from jax.experimental.pallas import tpu_sc as plsc

_NC = 2
_NS = 16
_NW = _NC * _NS
_CHUNK = 128


@functools.lru_cache(maxsize=None)
def _build(V, D, Dp, steps):
    mesh = plsc.VectorSubcoreMesh(
        core_axis_name="c", subcore_axis_name="s",
        num_cores=_NC, num_subcores=_NS)

    @functools.partial(
        pl.kernel,
        out_type=jax.ShapeDtypeStruct((_NW, steps, _CHUNK, Dp), jnp.float32),
        mesh=mesh,
        compiler_params=pltpu.CompilerParams(use_tc_tiling_on_sc=False),
        scratch_types=[
            pltpu.VMEM((_CHUNK,), jnp.int32),
            pltpu.VMEM((_CHUNK, Dp), jnp.float32),
            pltpu.SemaphoreType.DMA,
        ],
    )
    def _emb(idx_hbm, tab_hbm, out_hbm, idx_v, rows_v, sem0):
        wid = lax.axis_index("s") * _NC + lax.axis_index("c")

        def body(g, _):
            pltpu.sync_copy(idx_hbm.at[wid, g], idx_v)
            pltpu.async_copy(tab_hbm.at[idx_v], rows_v, sem0).wait()
            pltpu.sync_copy(rows_v, out_hbm.at[wid, g])
            return ()

        lax.fori_loop(0, steps, body, (), unroll=False)

    return _emb


def kernel(indices, table):
    Bb, Ll = indices.shape
    V, D = table.shape
    N = Bb * Ll
    assert N % (_NW * _CHUNK) == 0
    steps = N // (_NW * _CHUNK)
    Dp = (D + 7) // 8 * 8
    tab = table if Dp == D else jnp.pad(table, ((0, 0), (0, Dp - D)))
    idx = indices.reshape(_NW, steps, _CHUNK)
    out = _build(V, D, Dp, steps)(idx, tab)
    embeddings = out.reshape(N, Dp)[:, :D].reshape(Bb, Ll, D)
    mask = jnp.ones((Bb, Ll), dtype=jnp.int32)
    return embeddings, mask


# speedup vs baseline: 2.0260x; 2.0260x over previous
import functools

import jax
import jax.numpy as jnp
from jax import lax
from jax.experimental import pallas as pl
from jax.experimental.pallas import tpu as pltpu
from jax.experimental.pallas import tpu_sc as plsc

_NC = 2
_NS = 16
_NW = _NC * _NS
_CHUNK = 64
_LANES = 16


@functools.lru_cache(maxsize=None)
def _build(V, D, Dp, steps):
    mesh = plsc.VectorSubcoreMesh(
        core_axis_name="c", subcore_axis_name="s",
        num_cores=_NC, num_subcores=_NS)
    main = D - D % 128
    n_full = (D - main) // _LANES
    rem = D - main - n_full * _LANES

    @functools.partial(
        pl.kernel,
        out_type=jax.ShapeDtypeStruct(
            (_NW, steps, _CHUNK, D), jnp.float32),
        mesh=mesh,
        compiler_params=pltpu.CompilerParams(needs_layout_passes=False),
        scratch_types=[
            pltpu.VMEM((steps, _CHUNK), jnp.int32),
            pltpu.VMEM((2, _CHUNK, 128), jnp.float32),
            pltpu.VMEM((2, _CHUNK, D), jnp.float32),
            pltpu.SemaphoreType.DMA,
            pltpu.SemaphoreType.DMA,
            pltpu.SemaphoreType.DMA,
            pltpu.SemaphoreType.DMA,
        ],
    )
    def _emb(idx_hbm, tab_main, tab_tail, out_hbm, idx_v, tails, r300,
             gsem0, gsem1, osem0, osem1):
        wid = lax.axis_index("s") * _NC + lax.axis_index("c")
        gsems = (gsem0, gsem1)
        osems = (osem0, osem1)
        pltpu.sync_copy(idx_hbm.at[wid], idx_v)

        def start_gathers(g, b):
            pltpu.async_copy(
                tab_main.at[idx_v.at[g]], r300.at[b, :, pl.ds(0, main)],
                gsems[b])
            pltpu.async_copy(
                tab_tail.at[idx_v.at[g]], tails.at[b], gsems[b])

        def wait_gathers(g, b):
            pltpu.make_async_copy(
                tab_main.at[idx_v.at[g]], r300.at[b, :, pl.ds(0, main)],
                gsems[b]).wait()
            pltpu.make_async_copy(
                tab_tail.at[idx_v.at[g]], tails.at[b], gsems[b]).wait()

        start_gathers(0, 0)

        def pair(g0, _):
            for b in range(2):
                g = g0 * 2 + b
                nxt = g + 1

                @pl.when(nxt < steps)
                def _():
                    @pl.when(nxt >= 2)
                    def _():
                        pltpu.make_async_copy(
                            r300.at[1 - b], out_hbm.at[wid, g - 1],
                            osems[1 - b]).wait()

                    start_gathers(nxt, 1 - b)

                wait_gathers(g, b)

                def tail(i, _):
                    for k in range(n_full):
                        r300[b, i, pl.ds(main + k * _LANES, _LANES)] = (
                            tails[b, i, pl.ds(k * _LANES, _LANES)])
                    if rem:
                        base = main + n_full * _LANES
                        v = tails[b, i, pl.ds(n_full * _LANES, _LANES)]
                        plsc.store_scatter(
                            r300.at[b],
                            [jnp.full((_LANES,), i, jnp.int32),
                             base + lax.iota(jnp.int32, _LANES)],
                            v, mask=lax.iota(jnp.int32, _LANES) < rem)
                    return ()

                lax.fori_loop(0, _CHUNK, tail, (), unroll=8)

                pltpu.async_copy(r300.at[b], out_hbm.at[wid, g], osems[b])
            return ()

        lax.fori_loop(0, steps // 2, pair, (), unroll=False)
        for b in range(2):
            pltpu.make_async_copy(
                r300.at[b], out_hbm.at[wid, steps - 2 + b], osems[b]).wait()

    return _emb


def kernel(indices, table):
    Bb, Ll = indices.shape
    V, D = table.shape
    N = Bb * Ll
    assert N % (_NW * _CHUNK) == 0
    steps = N // (_NW * _CHUNK)
    main = D - D % 128
    tab_main = table[:, :main]
    tab_tail = jnp.pad(table[:, main:], ((0, 0), (0, 128 - (D - main))))
    idx = indices.reshape(_NW, steps, _CHUNK)
    out = _build(V, D, D, steps)(idx, tab_main, tab_tail)
    embeddings = out.reshape(Bb, Ll, D)
    mask = jnp.ones((Bb, Ll), dtype=jnp.int32)
    return embeddings, mask


# speedup vs baseline: 2.0455x; 1.0096x over previous
import functools

import jax
import jax.numpy as jnp
from jax import lax
from jax.experimental import pallas as pl
from jax.experimental.pallas import tpu as pltpu
from jax.experimental.pallas import tpu_sc as plsc

_NC = 2
_NS = 16
_NW = _NC * _NS
_CHUNK = 64
_LANES = 16


@functools.lru_cache(maxsize=None)
def _build(V, D, Dp, steps):
    mesh = plsc.VectorSubcoreMesh(
        core_axis_name="c", subcore_axis_name="s",
        num_cores=_NC, num_subcores=_NS)
    main = D - D % 128
    n_full = (D - main) // _LANES
    rem = D - main - n_full * _LANES

    @functools.partial(
        pl.kernel,
        out_type=jax.ShapeDtypeStruct(
            (_NW, steps, _CHUNK, D), jnp.float32),
        mesh=mesh,
        compiler_params=pltpu.CompilerParams(needs_layout_passes=False),
        scratch_types=[
            pltpu.VMEM((steps, _CHUNK), jnp.int32),
            pltpu.VMEM((2, _CHUNK, 128), jnp.float32),
            pltpu.VMEM((2, _CHUNK, D), jnp.float32),
            pltpu.SemaphoreType.DMA,
            pltpu.SemaphoreType.DMA,
            pltpu.SemaphoreType.DMA,
            pltpu.SemaphoreType.DMA,
        ],
    )
    def _emb(idx_hbm, tab_hbm, tab_tail, out_hbm, idx_v, tails, r300,
             gsem0, gsem1, osem0, osem1):
        wid = lax.axis_index("s") * _NC + lax.axis_index("c")
        gsems = (gsem0, gsem1)
        osems = (osem0, osem1)
        tab_main = tab_hbm.at[:, pl.ds(0, main)]
        pltpu.sync_copy(idx_hbm.at[wid], idx_v)

        def start_gathers(g, b):
            pltpu.async_copy(
                tab_main.at[idx_v.at[g]], r300.at[b, :, pl.ds(0, main)],
                gsems[b])
            pltpu.async_copy(
                tab_tail.at[idx_v.at[g]], tails.at[b], gsems[b])

        def wait_gathers(g, b):
            pltpu.make_async_copy(
                tab_main.at[idx_v.at[g]], r300.at[b, :, pl.ds(0, main)],
                gsems[b]).wait()
            pltpu.make_async_copy(
                tab_tail.at[idx_v.at[g]], tails.at[b], gsems[b]).wait()

        start_gathers(0, 0)

        def pair(g0, _):
            for b in range(2):
                g = g0 * 2 + b
                nxt = g + 1

                @pl.when(nxt < steps)
                def _():
                    @pl.when(nxt >= 2)
                    def _():
                        pltpu.make_async_copy(
                            r300.at[1 - b], out_hbm.at[wid, g - 1],
                            osems[1 - b]).wait()

                    start_gathers(nxt, 1 - b)

                wait_gathers(g, b)

                def tail(i, _):
                    for k in range(n_full):
                        r300[b, i, pl.ds(main + k * _LANES, _LANES)] = (
                            tails[b, i, pl.ds(k * _LANES, _LANES)])
                    if rem:
                        base = main + n_full * _LANES
                        v = tails[b, i, pl.ds(n_full * _LANES, _LANES)]
                        plsc.store_scatter(
                            r300.at[b],
                            [jnp.full((_LANES,), i, jnp.int32),
                             base + lax.iota(jnp.int32, _LANES)],
                            v, mask=lax.iota(jnp.int32, _LANES) < rem)
                    return ()

                lax.fori_loop(0, _CHUNK, tail, (), unroll=8)

                pltpu.async_copy(r300.at[b], out_hbm.at[wid, g], osems[b])
            return ()

        lax.fori_loop(0, steps // 2, pair, (), unroll=False)
        for b in range(2):
            pltpu.make_async_copy(
                r300.at[b], out_hbm.at[wid, steps - 2 + b], osems[b]).wait()

    return _emb


def kernel(indices, table):
    Bb, Ll = indices.shape
    V, D = table.shape
    N = Bb * Ll
    assert N % (_NW * _CHUNK) == 0
    steps = N // (_NW * _CHUNK)
    main = D - D % 128
    tab_tail = jnp.pad(table[:, main:], ((0, 0), (0, 128 - (D - main))))
    idx = indices.reshape(_NW, steps, _CHUNK)
    out = _build(V, D, D, steps)(idx, table, tab_tail)
    embeddings = out.reshape(Bb, Ll, D)
    mask = jnp.ones((Bb, Ll), dtype=jnp.int32)
    return embeddings, mask


# speedup vs baseline: 2.0459x; 1.0002x over previous
import functools

import jax
import jax.numpy as jnp
from jax import lax
from jax.experimental import pallas as pl
from jax.experimental.pallas import tpu as pltpu
from jax.experimental.pallas import tpu_sc as plsc

_NC = 2
_NS = 16
_NW = _NC * _NS
_CHUNK = 64
_LANES = 16


@functools.lru_cache(maxsize=None)
def _build(V, D, Dp, steps):
    mesh = plsc.VectorSubcoreMesh(
        core_axis_name="c", subcore_axis_name="s",
        num_cores=_NC, num_subcores=_NS)
    main = D - D % 128
    n_full = (D - main) // _LANES
    rem = D - main - n_full * _LANES

    @functools.partial(
        pl.kernel,
        out_type=jax.ShapeDtypeStruct(
            (_NW * steps * _CHUNK, D), jnp.float32),
        mesh=mesh,
        compiler_params=pltpu.CompilerParams(needs_layout_passes=False),
        scratch_types=[
            pltpu.VMEM((steps, _CHUNK), jnp.int32),
            pltpu.VMEM((2, _CHUNK, 128), jnp.float32),
            pltpu.VMEM((2, _CHUNK, D), jnp.float32),
            pltpu.SemaphoreType.DMA,
            pltpu.SemaphoreType.DMA,
            pltpu.SemaphoreType.DMA,
            pltpu.SemaphoreType.DMA,
        ],
    )
    def _emb(idx_hbm, tab_hbm, tab_tail, out_hbm, idx_v, tails, r300,
             gsem0, gsem1, osem0, osem1):
        wid = lax.axis_index("s") * _NC + lax.axis_index("c")
        gsems = (gsem0, gsem1)
        osems = (osem0, osem1)
        tab_main = tab_hbm.at[:, pl.ds(0, main)]
        base = wid * (steps * _CHUNK)
        pltpu.sync_copy(idx_hbm.at[wid], idx_v)

        def start_gathers(g, b):
            pltpu.async_copy(
                tab_main.at[idx_v.at[g]], r300.at[b, :, pl.ds(0, main)],
                gsems[b])
            pltpu.async_copy(
                tab_tail.at[idx_v.at[g]], tails.at[b], gsems[b])

        def wait_gathers(g, b):
            pltpu.make_async_copy(
                tab_main.at[idx_v.at[g]], r300.at[b, :, pl.ds(0, main)],
                gsems[b]).wait()
            pltpu.make_async_copy(
                tab_tail.at[idx_v.at[g]], tails.at[b], gsems[b]).wait()

        start_gathers(0, 0)

        def pair(g0, _):
            for b in range(2):
                g = g0 * 2 + b
                nxt = g + 1

                @pl.when(nxt < steps)
                def _():
                    @pl.when(nxt >= 2)
                    def _():
                        pltpu.make_async_copy(
                            r300.at[1 - b],
                            out_hbm.at[pl.ds(base + (g - 1) * _CHUNK,
                                             _CHUNK)],
                            osems[1 - b]).wait()

                    start_gathers(nxt, 1 - b)

                wait_gathers(g, b)

                def tail(i, _):
                    for k in range(n_full):
                        r300[b, i, pl.ds(main + k * _LANES, _LANES)] = (
                            tails[b, i, pl.ds(k * _LANES, _LANES)])
                    if rem:
                        base = main + n_full * _LANES
                        v = tails[b, i, pl.ds(n_full * _LANES, _LANES)]
                        plsc.store_scatter(
                            r300.at[b],
                            [jnp.full((_LANES,), i, jnp.int32),
                             base + lax.iota(jnp.int32, _LANES)],
                            v, mask=lax.iota(jnp.int32, _LANES) < rem)
                    return ()

                lax.fori_loop(0, _CHUNK, tail, (), unroll=8)

                pltpu.async_copy(
                    r300.at[b],
                    out_hbm.at[pl.ds(base + g * _CHUNK, _CHUNK)], osems[b])
            return ()

        lax.fori_loop(0, steps // 2, pair, (), unroll=False)
        for b in range(2):
            pltpu.make_async_copy(
                r300.at[b],
                out_hbm.at[pl.ds(base + (steps - 2 + b) * _CHUNK, _CHUNK)],
                osems[b]).wait()

    return _emb


def kernel(indices, table):
    Bb, Ll = indices.shape
    V, D = table.shape
    N = Bb * Ll
    assert N % (_NW * _CHUNK) == 0
    steps = N // (_NW * _CHUNK)
    main = D - D % 128
    tab_tail = jnp.pad(table[:, main:], ((0, 0), (0, 128 - (D - main))))
    idx = indices.reshape(_NW, steps, _CHUNK)
    out = _build(V, D, D, steps)(idx, table, tab_tail)
    embeddings = out.reshape(Bb, Ll, D)
    mask = jnp.ones((Bb, Ll), dtype=jnp.int32)
    return embeddings, mask
